# baseline (speedup 1.0000x reference)
import jax
import jax.numpy as jnp
from jax.experimental import pallas as pl
from jax.experimental.pallas import tpu as pltpu
from jax.experimental.pallas import tpu_sc as plsc

_BETA = 0.2
_K = 512
_C = 32
_TN = 2048


def _tc_body(xp_ref, a2_ref, b2_ref, cbt2_ref, cbt_ref,
             z_ref, qt_ref, ls_ref):
    i = pl.program_id(0)
    xt = xp_ref[0]
    a2t = a2_ref[0]
    b2c = b2_ref[...]
    cbt2 = cbt2_ref[...]
    cbt = cbt_ref[...]
    ab2 = jnp.dot(xt, cbt2, preferred_element_type=jnp.float32)
    d2 = jnp.maximum((a2t + b2c) + ab2.T, 0.0)
    d = jnp.sqrt(d2)
    m = jnp.min(d, axis=0, keepdims=True)
    iota = jax.lax.broadcasted_iota(jnp.int32, (_K, _TN), 0)
    z = jnp.min(jnp.where(d == m, iota, _K), axis=0)
    z_ref[0, 0, :] = z
    onehot = jnp.where(iota == z[None, :], 1.0, 0.0)
    qt = jnp.dot(cbt, onehot, preferred_element_type=jnp.float32)
    qt_ref[0] = qt.reshape(_C, _TN // 64, 64)

    part = jnp.sum(m * m).reshape(1, 1)

    @pl.when(i == 0)
    def _():
        ls_ref[...] = jnp.zeros_like(ls_ref)

    ls_ref[...] += part


def _tc_call(xp, a2r, b2c, cbt2, cbt, b_start, b_count, N):
    nt = N // _TN
    grid = (b_count * nt,)
    off = b_start * nt
    out_shapes = [
        jax.ShapeDtypeStruct((b_count * nt, 1, _TN), jnp.int32),
        jax.ShapeDtypeStruct((b_count, _C, N // 64, 64), jnp.float32),
        jax.ShapeDtypeStruct((1, 1), jnp.float32),
    ]
    return pl.pallas_call(
        _tc_body,
        grid=grid,
        in_specs=[
            pl.BlockSpec((1, _TN, _C),
                         lambda i: (b_start + i // nt, i % nt, 0)),
            pl.BlockSpec((1, 1, _TN), lambda i: (off + i, 0, 0)),
            pl.BlockSpec((_K, 1), lambda i: (0, 0)),
            pl.BlockSpec((_C, _K), lambda i: (0, 0)),
            pl.BlockSpec((_C, _K), lambda i: (0, 0)),
        ],
        out_specs=[
            pl.BlockSpec((1, 1, _TN), lambda i: (i, 0, 0)),
            pl.BlockSpec((1, _C, _TN // 64, 64),
                         lambda i: (i // nt, 0, i % nt, 0)),
            pl.BlockSpec((1, 1), lambda i: (0, 0)),
        ],
        out_shape=out_shapes,
    )(xp, a2r, b2c, cbt2, cbt)


def _sc_gather(cb_pad, idx_flat, n_idx):
    window = 128
    row = 128
    idx2d = idx_flat.reshape(1, n_idx)

    @pl.kernel(
        out_type=jax.ShapeDtypeStruct((n_idx, _C), jnp.float32),
        mesh=plsc.VectorSubcoreMesh(core_axis_name="c", subcore_axis_name="s"),
        scratch_types=[pltpu.VMEM((window, row), jnp.float32)],
    )
    def _gather_kernel(cb_hbm, i_hbm, o_hbm, tmp_ref):
        def body(i_vmem, o_vmem):
            pltpu.sync_copy(cb_hbm.at[i_vmem.at[0]], tmp_ref)

            @pl.loop(0, window)
            def _(r):
                @pl.loop(0, _C, step=16)
                def _(c):
                    o_vmem[r, pl.ds(c, 16)] = tmp_ref[r, pl.ds(c, 16)]

        pltpu.emit_pipeline(
            body,
            grid=(n_idx // window,),
            in_specs=[pl.BlockSpec((1, window), index_map=lambda i: (0, i))],
            out_specs=[pl.BlockSpec((window, _C), index_map=lambda i: (i, 0))],
            core_axis_name=("c", "s"),
            dimension_semantics=(pltpu.PARALLEL,),
        )(i_hbm, o_hbm)

    return _gather_kernel(cb_pad, idx2d)


def kernel(x, codebook):
    B, C, H, W = x.shape
    N = H * W
    xp = jnp.transpose(x, (0, 2, 3, 1)).reshape(B, -1, C)
    a2 = jnp.sum(xp * xp, axis=-1, keepdims=True)
    cb_b = jnp.broadcast_to(codebook[None, :, :], (B, _K, C))
    b2 = jnp.sum(cb_b * cb_b, axis=-1)[:, None, :]
    b2c = b2[0].reshape(_K, 1)
    cbt = codebook.T
    cbt2 = -2.0 * cbt
    nt = N // _TN
    a2r = a2.reshape(B * nt, 1, _TN)

    cb_pad = jnp.pad(codebook, ((0, 0), (0, 128 - _C)))
    hb = B // 2
    z1, q1, l1 = _tc_call(xp, a2r, b2c, cbt2, cbt, 0, hb, N)
    z2, q2, l2 = _tc_call(xp, a2r, b2c, cbt2, cbt, hb, hb, N)
    f1 = _sc_gather(cb_pad, z1.reshape(hb * N), hb * N)
    f2 = _sc_gather(cb_pad, z2.reshape(hb * N), hb * N)
    min_index_out = jnp.concatenate([f1, f2], axis=0).reshape(B, C, H, W)
    quantized = jnp.concatenate([q1, q2], axis=0)
    loss = (l1[0, 0] + l2[0, 0]) / jnp.float32(B * N * C)
    codebook_loss = loss
    commitment_loss = loss
    quantizer_loss = _BETA * commitment_loss + codebook_loss
    return (quantized, codebook_loss, commitment_loss, quantizer_loss,
            min_index_out)


# speedup vs baseline: 1.0232x; 1.0232x over previous
import jax
import jax.numpy as jnp
from jax.experimental import pallas as pl
from jax.experimental.pallas import tpu as pltpu
from jax.experimental.pallas import tpu_sc as plsc

_BETA = 0.2
_K = 512
_C = 32
_TN = 2048


def _tc_body(xp_ref, a2_ref, b2_ref, cbt2_ref, cbt_ref,
             z_ref, qt_ref, ls_ref):
    i = pl.program_id(0)
    xt = xp_ref[0]
    a2t = a2_ref[0]
    b2c = b2_ref[...]
    cbt2 = cbt2_ref[...]
    cbt = cbt_ref[...]
    ab2 = jnp.dot(xt, cbt2, preferred_element_type=jnp.float32)
    d2 = jnp.maximum((a2t + b2c) + ab2.T, 0.0)
    d = jnp.sqrt(d2)
    m = jnp.min(d, axis=0, keepdims=True)
    iota = jax.lax.broadcasted_iota(jnp.int32, (_K, _TN), 0)
    z = jnp.min(jnp.where(d == m, iota, _K), axis=0)
    z_ref[0, 0, :] = z
    onehot = jnp.where(iota == z[None, :], 1.0, 0.0)
    qt_ref[0] = jnp.dot(cbt, onehot, preferred_element_type=jnp.float32)

    part = jnp.sum(m * m).reshape(1, 1)

    @pl.when(i == 0)
    def _():
        ls_ref[...] = jnp.zeros_like(ls_ref)

    ls_ref[...] += part


def _tc_call(xp, a2r, b2c, cbt2, cbt, B, N):
    nt = N // _TN
    grid = (B * nt,)
    out_shapes = [
        jax.ShapeDtypeStruct((B * nt, 1, _TN), jnp.int32),
        jax.ShapeDtypeStruct((B, _C, N), jnp.float32),
        jax.ShapeDtypeStruct((1, 1), jnp.float32),
    ]
    return pl.pallas_call(
        _tc_body,
        grid=grid,
        in_specs=[
            pl.BlockSpec((1, _TN, _C), lambda i: (i // nt, i % nt, 0)),
            pl.BlockSpec((1, 1, _TN), lambda i: (i, 0, 0)),
            pl.BlockSpec((_K, 1), lambda i: (0, 0)),
            pl.BlockSpec((_C, _K), lambda i: (0, 0)),
            pl.BlockSpec((_C, _K), lambda i: (0, 0)),
        ],
        out_specs=[
            pl.BlockSpec((1, 1, _TN), lambda i: (i, 0, 0)),
            pl.BlockSpec((1, _C, _TN), lambda i: (i // nt, 0, i % nt)),
            pl.BlockSpec((1, 1), lambda i: (0, 0)),
        ],
        out_shape=out_shapes,
    )(xp, a2r, b2c, cbt2, cbt)


def _sc_gather(cb_pad, idx_flat, n_idx):
    window = 128
    row = 128
    idx2d = idx_flat.reshape(1, n_idx)

    @pl.kernel(
        out_type=jax.ShapeDtypeStruct((n_idx, row), jnp.float32),
        mesh=plsc.VectorSubcoreMesh(core_axis_name="c", subcore_axis_name="s"),
    )
    def _gather_kernel(cb_hbm, i_hbm, o_hbm):
        def body(i_vmem, o_vmem):
            pltpu.sync_copy(cb_hbm.at[i_vmem.at[0]], o_vmem)

        pltpu.emit_pipeline(
            body,
            grid=(n_idx // window,),
            in_specs=[pl.BlockSpec((1, window), index_map=lambda i: (0, i))],
            out_specs=[pl.BlockSpec((window, row), index_map=lambda i: (i, 0))],
            core_axis_name=("c", "s"),
            dimension_semantics=(pltpu.PARALLEL,),
        )(i_hbm, o_hbm)

    return _gather_kernel(cb_pad, idx2d)


def kernel(x, codebook):
    B, C, H, W = x.shape
    N = H * W
    xp = jnp.transpose(x, (0, 2, 3, 1)).reshape(B, -1, C)
    a2 = jnp.sum(xp * xp, axis=-1, keepdims=True)
    cb_b = jnp.broadcast_to(codebook[None, :, :], (B, _K, C))
    b2 = jnp.sum(cb_b * cb_b, axis=-1)[:, None, :]
    b2c = b2[0].reshape(_K, 1)
    cbt = codebook.T
    cbt2 = -2.0 * cbt
    nt = N // _TN
    a2r = a2.reshape(B * nt, 1, _TN)

    cb_pad = jnp.pad(codebook, ((0, 0), (0, 128 - _C)))
    z3d, qt, lsum = _tc_call(xp, a2r, b2c, cbt2, cbt, B, N)
    flat = _sc_gather(cb_pad, z3d.reshape(B * N), B * N)
    min_index_out = flat[:, :_C].reshape(B, C, H, W)
    quantized = qt.reshape(B, C, H, W)
    loss = lsum[0, 0] / jnp.float32(B * N * C)
    codebook_loss = loss
    commitment_loss = loss
    quantizer_loss = _BETA * commitment_loss + codebook_loss
    return (quantized, codebook_loss, commitment_loss, quantizer_loss,
            min_index_out)


# speedup vs baseline: 1.0497x; 1.0259x over previous
import jax
import jax.numpy as jnp
from jax.experimental import pallas as pl
from jax.experimental.pallas import tpu as pltpu
from jax.experimental.pallas import tpu_sc as plsc

_BETA = 0.2
_K = 512
_C = 32
_TN = 4096


def _tc_body(xp_ref, a2_ref, b2_ref, cbt2_ref, cbt_ref,
             z_ref, qt_ref, ls_ref):
    i = pl.program_id(0)
    xt = xp_ref[0]
    a2t = a2_ref[0]
    b2c = b2_ref[...]
    cbt2 = cbt2_ref[...]
    cbt = cbt_ref[...]
    ab2 = jnp.dot(xt, cbt2, preferred_element_type=jnp.float32)
    d2 = jnp.maximum((a2t + b2c) + ab2.T, 0.0)
    d = jnp.sqrt(d2)
    m = jnp.min(d, axis=0, keepdims=True)
    iota = jax.lax.broadcasted_iota(jnp.int32, (_K, _TN), 0)
    z = jnp.min(jnp.where(d == m, iota, _K), axis=0)
    z_ref[0, 0, :] = z
    onehot = jnp.where(iota == z[None, :], 1.0, 0.0)
    qt_ref[0] = jnp.dot(cbt, onehot, preferred_element_type=jnp.float32)

    part = jnp.sum(m * m).reshape(1, 1)

    @pl.when(i == 0)
    def _():
        ls_ref[...] = jnp.zeros_like(ls_ref)

    ls_ref[...] += part


def _tc_call(xp, a2r, b2c, cbt2, cbt, B, N):
    nt = N // _TN
    grid = (B * nt,)
    out_shapes = [
        jax.ShapeDtypeStruct((B * nt, 1, _TN), jnp.int32),
        jax.ShapeDtypeStruct((B, _C, N), jnp.float32),
        jax.ShapeDtypeStruct((1, 1), jnp.float32),
    ]
    return pl.pallas_call(
        _tc_body,
        grid=grid,
        in_specs=[
            pl.BlockSpec((1, _TN, _C), lambda i: (i // nt, i % nt, 0)),
            pl.BlockSpec((1, 1, _TN), lambda i: (i, 0, 0)),
            pl.BlockSpec((_K, 1), lambda i: (0, 0)),
            pl.BlockSpec((_C, _K), lambda i: (0, 0)),
            pl.BlockSpec((_C, _K), lambda i: (0, 0)),
        ],
        out_specs=[
            pl.BlockSpec((1, 1, _TN), lambda i: (i, 0, 0)),
            pl.BlockSpec((1, _C, _TN), lambda i: (i // nt, 0, i % nt)),
            pl.BlockSpec((1, 1), lambda i: (0, 0)),
        ],
        out_shape=out_shapes,
    )(xp, a2r, b2c, cbt2, cbt)


def _sc_gather(cb_pad, idx_flat, n_idx):
    window = 128
    row = 128
    idx2d = idx_flat.reshape(1, n_idx)

    @pl.kernel(
        out_type=jax.ShapeDtypeStruct((n_idx, row), jnp.float32),
        mesh=plsc.VectorSubcoreMesh(core_axis_name="c", subcore_axis_name="s"),
    )
    def _gather_kernel(cb_hbm, i_hbm, o_hbm):
        def body(i_vmem, o_vmem):
            pltpu.sync_copy(cb_hbm.at[i_vmem.at[0]], o_vmem)

        pltpu.emit_pipeline(
            body,
            grid=(n_idx // window,),
            in_specs=[pl.BlockSpec((1, window), index_map=lambda i: (0, i))],
            out_specs=[pl.BlockSpec((window, row), index_map=lambda i: (i, 0))],
            core_axis_name=("c", "s"),
            dimension_semantics=(pltpu.PARALLEL,),
        )(i_hbm, o_hbm)

    return _gather_kernel(cb_pad, idx2d)


def kernel(x, codebook):
    B, C, H, W = x.shape
    N = H * W
    xp = jnp.transpose(x, (0, 2, 3, 1)).reshape(B, -1, C)
    a2 = jnp.sum(xp * xp, axis=-1, keepdims=True)
    cb_b = jnp.broadcast_to(codebook[None, :, :], (B, _K, C))
    b2 = jnp.sum(cb_b * cb_b, axis=-1)[:, None, :]
    b2c = b2[0].reshape(_K, 1)
    cbt = codebook.T
    cbt2 = -2.0 * cbt
    nt = N // _TN
    a2r = a2.reshape(B * nt, 1, _TN)

    cb_pad = jnp.pad(codebook, ((0, 0), (0, 128 - _C)))
    z3d, qt, lsum = _tc_call(xp, a2r, b2c, cbt2, cbt, B, N)
    flat = _sc_gather(cb_pad, z3d.reshape(B * N), B * N)
    min_index_out = flat[:, :_C].reshape(B, C, H, W)
    quantized = qt.reshape(B, C, H, W)
    loss = lsum[0, 0] / jnp.float32(B * N * C)
    codebook_loss = loss
    commitment_loss = loss
    quantizer_loss = _BETA * commitment_loss + codebook_loss
    return (quantized, codebook_loss, commitment_loss, quantizer_loss,
            min_index_out)
